# baseline (speedup 1.0000x reference)
import functools

import jax
import jax.numpy as jnp
from jax import lax
from jax.experimental import pallas as pl
from jax.experimental.pallas import tpu as pltpu
from jax.experimental.pallas import tpu_sc as plsc

B = 16384
V = 100000
D = 64
K = 20
NC = 2
NS = 16
NW = NC * NS
BPW = B // NW
CH = 32
NCH = BPW // CH
COLS = 32
PAD_VAL = -30.0 / 16.0
LBH = 2048
NBLK = 25
HALF = NBLK * LBH


def _tc_pack_table(wt):
    def body(x1_ref, x2_ref, o_ref):
        y1 = jnp.transpose(x1_ref[...], (1, 0))
        y2 = jnp.transpose(x2_ref[...], (1, 0))
        o_ref[...] = jnp.concatenate([y1, y2], axis=1).astype(jnp.bfloat16)

    last_blk = (V - LBH) // LBH

    return pl.pallas_call(
        body,
        grid=(NBLK,),
        in_specs=[
            pl.BlockSpec((D, LBH), lambda i: (0, i)),
            pl.BlockSpec((D, LBH),
                         lambda i: (0, jnp.minimum(i + NBLK, last_blk))),
        ],
        out_specs=pl.BlockSpec((LBH, 2 * D), lambda i: (i, 0)),
        out_shape=jax.ShapeDtypeStruct((HALF, 2 * D), jnp.bfloat16),
    )(wt, wt)


def _sc_scores(cen, pos, neg_t, table):
    mesh = plsc.VectorSubcoreMesh(core_axis_name="c", subcore_axis_name="s")

    @functools.partial(
        pl.kernel,
        out_type=jax.ShapeDtypeStruct((B, COLS), jnp.float32),
        mesh=mesh,
        scratch_types=[
            pltpu.VMEM((BPW,), jnp.int32),
            pltpu.VMEM((BPW,), jnp.int32),
            pltpu.VMEM((K, BPW), jnp.int32),
            pltpu.VMEM((2, CH, D), jnp.bfloat16),
            pltpu.VMEM((2, CH, D), jnp.bfloat16),
            pltpu.VMEM((2, CH * K, D), jnp.bfloat16),
            pltpu.VMEM((2, CH, COLS), jnp.float32),
            pltpu.SemaphoreType.DMA,
            pltpu.SemaphoreType.DMA,
        ],
        compiler_params=pltpu.CompilerParams(
            needs_layout_passes=False, use_tc_tiling_on_sc=False),
    )
    def k(cen_hbm, pos_hbm, neg_hbm, w_hbm, out_hbm,
          cen_i, pos_i, neg_i, cen_r, pos_r, neg_r, sc_v, sem0, sem1):
        wid = lax.axis_index("s") * NC + lax.axis_index("c")
        pltpu.sync_copy(cen_hbm.at[pl.ds(wid * BPW, BPW)], cen_i)
        pltpu.sync_copy(pos_hbm.at[pl.ds(wid * BPW, BPW)], pos_i)
        for kk in range(K):
            pltpu.sync_copy(neg_hbm.at[kk, pl.ds(wid * BPW, BPW)],
                            neg_i.at[kk])

        def fix(v):
            return v + v - jnp.where(v >= HALF, 2 * HALF - 1,
                                     0).astype(jnp.int32)

        for i in range(BPW // 16):
            sl = pl.ds(i * 16, 16)
            cen_i[sl] = fix(cen_i[sl])
            pos_i[sl] = fix(pos_i[sl])

        def fixrow(kk, carry):
            for i in range(BPW // 16):
                sl = pl.ds(i * 16, 16)
                neg_i[kk, sl] = fix(neg_i[kk, sl])
            return carry

        lax.fori_loop(0, K, fixrow, 0)

        lanes = lax.iota(jnp.int32, 16)
        rot_idx = {s: (lanes + s) & 15 for s in (1, 2, 4, 8, 15, 14, 12)}
        sign0 = jnp.where(lanes == 0, 1.0, -1.0).astype(jnp.float32)

        def rot(v, s):
            if isinstance(v, float):
                return v
            return v.at[rot_idx[s]].get(mode="promise_in_bounds")

        def tree16(vs):
            w = 16
            while len(vs) > 1:
                s = w // 2
                mask = (lanes & (w - 1)) < s
                nxt = []
                for a, b in zip(vs[0::2], vs[1::2]):
                    lo = a + rot(a, s)
                    hi = b + rot(b, 16 - s)
                    if isinstance(lo, float) and isinstance(hi, float):
                        assert lo == hi
                        nxt.append(lo)
                    else:
                        nxt.append(jnp.where(mask, lo, hi))
                vs = nxt
                w = s
            return vs[0]

        sems = [sem0, sem1]

        def copies(g, par):
            return [
                pltpu.make_async_copy(
                    w_hbm.at[cen_i.at[pl.ds(g * CH, CH)]],
                    cen_r.at[par], sems[par]),
                pltpu.make_async_copy(
                    w_hbm.at[pos_i.at[pl.ds(g * CH, CH)]],
                    pos_r.at[par], sems[par]),
            ] + [
                pltpu.make_async_copy(
                    w_hbm.at[neg_i.at[kk, pl.ds(g * CH, CH)]],
                    neg_r.at[par, pl.ds(kk * CH, CH)], sems[par])
                for kk in range(K)
            ]

        for cp in copies(0, 0):
            cp.start()

        def outer(o, carry):
            for par in (0, 1):
                g = o * 2 + par
                for cp in copies(g, par):
                    cp.wait()

                @pl.when(g + 1 < NCH)
                def _():
                    for cp in copies(g + 1, 1 - par):
                        cp.start()

                def unpack_row(rows, r):
                    out = []
                    for half in (0, 32):
                        w = plsc.bitcast(rows[par, r, pl.ds(half, 32)],
                                         jnp.int32)
                        out.append(plsc.bitcast(w << 16, jnp.float32))
                        out.append(plsc.bitcast(
                            w & jnp.int32(-65536), jnp.float32))
                    return out

                @plsc.parallel_loop(0, CH, unroll=2)
                def body(b):
                    c = unpack_row(cen_r, b)
                    p = unpack_row(pos_r, b)

                    def dot(vecs):
                        return (c[0] * vecs[0] + c[1] * vecs[1]
                                + c[2] * vecs[2] + c[3] * vecs[3])

                    t = [dot(p)]
                    t += [dot(unpack_row(neg_r, kk * CH + b))
                          for kk in range(K)]
                    sc_v[par, b, pl.ds(0, 16)] = tree16(t[:16]) * sign0
                    sc_v[par, b, pl.ds(16, 16)] = -tree16(
                        t[16:] + [PAD_VAL] * 11)

                pltpu.sync_copy(sc_v.at[par],
                                out_hbm.at[pl.ds(wid * BPW + g * CH, CH)])
            return carry

        lax.fori_loop(0, NCH // 2, outer, 0)

    return k(cen, pos, neg_t, table)


def _tc_loss(scores2d):
    def body(s_ref, o_ref):
        x = s_ref[...]
        o_ref[...] = (-jnp.sum(jax.nn.log_sigmoid(x)) / B).reshape(1, 1)

    out = pl.pallas_call(
        body,
        out_shape=jax.ShapeDtypeStruct((1, 1), jnp.float32),
    )(scores2d)
    return out[0, 0]


def kernel(centrals_words, pos_context, neg_context, W):
    cen = centrals_words.astype(jnp.int32)
    pos = pos_context.astype(jnp.int32)
    neg_t = neg_context.astype(jnp.int32).T
    table = _tc_pack_table(W.T).reshape(2 * HALF, D)
    scores = _sc_scores(cen, pos, neg_t, table)
    return _tc_loss(scores.reshape(B * COLS // 128, 128))


# speedup vs baseline: 2.2514x; 2.2514x over previous
import functools

import jax
import jax.numpy as jnp
from jax import lax
from jax.experimental import pallas as pl
from jax.experimental.pallas import tpu as pltpu
from jax.experimental.pallas import tpu_sc as plsc

B = 16384
V = 100000
D = 64
K = 20
NC = 2
NS = 16
NW = NC * NS
BPW = B // NW
CH = 32
NCH = BPW // CH
COLS = 32
PAD_VAL = -30.0 / 16.0
LBH = 2048
NBLK = 25
HALF = NBLK * LBH


def _tc_pack_table(wt):
    def body(x1_ref, x2_ref, o_ref):
        y1 = jnp.transpose(x1_ref[...], (1, 0))
        y2 = jnp.transpose(x2_ref[...], (1, 0))
        o_ref[...] = jnp.concatenate([y1, y2], axis=1)

    last_blk = (V - LBH) // LBH

    return pl.pallas_call(
        body,
        grid=(NBLK,),
        in_specs=[
            pl.BlockSpec((D, LBH), lambda i: (0, i)),
            pl.BlockSpec((D, LBH),
                         lambda i: (0, jnp.minimum(i + NBLK, last_blk))),
        ],
        out_specs=pl.BlockSpec((LBH, 2 * D), lambda i: (i, 0)),
        out_shape=jax.ShapeDtypeStruct((HALF, 2 * D), jnp.float32),
    )(wt, wt)


def _sc_scores(cen, pos, neg_t, table):
    mesh = plsc.VectorSubcoreMesh(core_axis_name="c", subcore_axis_name="s")

    @functools.partial(
        pl.kernel,
        out_type=jax.ShapeDtypeStruct((B, COLS), jnp.float32),
        mesh=mesh,
        scratch_types=[
            pltpu.VMEM((BPW,), jnp.int32),
            pltpu.VMEM((BPW,), jnp.int32),
            pltpu.VMEM((K, BPW), jnp.int32),
            pltpu.VMEM((2, CH, D), jnp.float32),
            pltpu.VMEM((2, CH, D), jnp.float32),
            pltpu.VMEM((2, CH * K, D), jnp.float32),
            pltpu.VMEM((2, CH, COLS), jnp.float32),
            pltpu.SemaphoreType.DMA,
            pltpu.SemaphoreType.DMA,
        ],
        compiler_params=pltpu.CompilerParams(
            needs_layout_passes=False, use_tc_tiling_on_sc=False),
    )
    def k(cen_hbm, pos_hbm, neg_hbm, w_hbm, out_hbm,
          cen_i, pos_i, neg_i, cen_r, pos_r, neg_r, sc_v, sem0, sem1):
        wid = lax.axis_index("s") * NC + lax.axis_index("c")
        pltpu.sync_copy(cen_hbm.at[pl.ds(wid * BPW, BPW)], cen_i)
        pltpu.sync_copy(pos_hbm.at[pl.ds(wid * BPW, BPW)], pos_i)
        for kk in range(K):
            pltpu.sync_copy(neg_hbm.at[kk, pl.ds(wid * BPW, BPW)],
                            neg_i.at[kk])

        def fix(v):
            return v + v - jnp.where(v >= HALF, 2 * HALF - 1,
                                     0).astype(jnp.int32)

        for i in range(BPW // 16):
            sl = pl.ds(i * 16, 16)
            cen_i[sl] = fix(cen_i[sl])
            pos_i[sl] = fix(pos_i[sl])

        def fixrow(kk, carry):
            for i in range(BPW // 16):
                sl = pl.ds(i * 16, 16)
                neg_i[kk, sl] = fix(neg_i[kk, sl])
            return carry

        lax.fori_loop(0, K, fixrow, 0)

        lanes = lax.iota(jnp.int32, 16)
        rot_idx = {s: (lanes + s) & 15 for s in (1, 2, 4, 8, 15, 14, 12)}
        sign0 = jnp.where(lanes == 0, 1.0, -1.0).astype(jnp.float32)

        def rot(v, s):
            if isinstance(v, float):
                return v
            return v.at[rot_idx[s]].get(mode="promise_in_bounds")

        def tree16(vs):
            w = 16
            while len(vs) > 1:
                s = w // 2
                mask = (lanes & (w - 1)) < s
                nxt = []
                for a, b in zip(vs[0::2], vs[1::2]):
                    lo = a + rot(a, s)
                    hi = b + rot(b, 16 - s)
                    if isinstance(lo, float) and isinstance(hi, float):
                        assert lo == hi
                        nxt.append(lo)
                    else:
                        nxt.append(jnp.where(mask, lo, hi))
                vs = nxt
                w = s
            return vs[0]

        sems = [sem0, sem1]

        def copies(g, par):
            return [
                pltpu.make_async_copy(
                    w_hbm.at[cen_i.at[pl.ds(g * CH, CH)]],
                    cen_r.at[par], sems[par]),
                pltpu.make_async_copy(
                    w_hbm.at[pos_i.at[pl.ds(g * CH, CH)]],
                    pos_r.at[par], sems[par]),
            ] + [
                pltpu.make_async_copy(
                    w_hbm.at[neg_i.at[kk, pl.ds(g * CH, CH)]],
                    neg_r.at[par, pl.ds(kk * CH, CH)], sems[par])
                for kk in range(K)
            ]

        for cp in copies(0, 0):
            cp.start()

        def outer(o, carry):
            for par in (0, 1):
                g = o * 2 + par
                for cp in copies(g, par):
                    cp.wait()

                @pl.when(g + 1 < NCH)
                def _():
                    for cp in copies(g + 1, 1 - par):
                        cp.start()

                @plsc.parallel_loop(0, CH, unroll=2)
                def body(b):
                    c = [cen_r[par, b, pl.ds(i * 16, 16)] for i in range(4)]
                    p = [pos_r[par, b, pl.ds(i * 16, 16)] for i in range(4)]

                    def dot(rows, r):
                        return (c[0] * rows[par, r, pl.ds(0, 16)]
                                + c[1] * rows[par, r, pl.ds(16, 16)]
                                + c[2] * rows[par, r, pl.ds(32, 16)]
                                + c[3] * rows[par, r, pl.ds(48, 16)])

                    t = [c[0] * p[0] + c[1] * p[1] + c[2] * p[2]
                         + c[3] * p[3]]
                    t += [dot(neg_r, kk * CH + b) for kk in range(K)]
                    sc_v[par, b, pl.ds(0, 16)] = tree16(t[:16]) * sign0
                    sc_v[par, b, pl.ds(16, 16)] = -tree16(
                        t[16:] + [PAD_VAL] * 11)

                pltpu.sync_copy(sc_v.at[par],
                                out_hbm.at[pl.ds(wid * BPW + g * CH, CH)])
            return carry

        lax.fori_loop(0, NCH // 2, outer, 0)

    return k(cen, pos, neg_t, table)


def _tc_loss(scores2d):
    def body(s_ref, o_ref):
        x = s_ref[...]
        o_ref[...] = (-jnp.sum(jax.nn.log_sigmoid(x)) / B).reshape(1, 1)

    out = pl.pallas_call(
        body,
        out_shape=jax.ShapeDtypeStruct((1, 1), jnp.float32),
    )(scores2d)
    return out[0, 0]


def kernel(centrals_words, pos_context, neg_context, W):
    cen = centrals_words.astype(jnp.int32)
    pos = pos_context.astype(jnp.int32)
    neg_t = neg_context.astype(jnp.int32).T
    table = _tc_pack_table(W.T).reshape(2 * HALF, D)
    scores = _sc_scores(cen, pos, neg_t, table)
    return _tc_loss(scores.reshape(B * COLS // 128, 128))
